# Initial kernel scaffold; baseline (speedup 1.0000x reference)
import jax
import jax.numpy as jnp
from jax.experimental import pallas as pl


def kernel(outputs, batch_y):
    raise NotImplementedError("write your pallas kernel here")


# speedup vs baseline: 9.2740x; 9.2740x over previous
import jax
import jax.numpy as jnp
from jax.experimental import pallas as pl

_B = 32
_L = 2101
_C = 64
_S = 64
_NB = 32
_W = _NB * _S
_PAD = _L - _W


def _fc_kernel(o_ref, y_ref, out_ref):
    o = o_ref[0]
    y = y_ref[0]
    d = o - y
    sq = d * d
    main = sq[:_W].reshape(_NB, _S, _C)
    s = jnp.sum(main, axis=1)
    tail = jnp.sum(sq[_W:], axis=0, keepdims=True)
    mp = s[:-1] + s[1:]
    nz = (mp != 0).astype(jnp.float32)
    num = jnp.concatenate([mp[:1], mp[:-1] + mp[1:], mp[-1:]], axis=0)
    cnt = jnp.concatenate([nz[:1], nz[:-1] + nz[1:], nz[-1:]], axis=0)
    v = num / cnt
    body = jnp.broadcast_to(v[:, None, :], (_NB, _S, _C)).reshape(_W, _C)
    tail_b = jnp.broadcast_to(tail, (_PAD, _C))
    out_ref[0] = jnp.concatenate([body, tail_b], axis=0)


def kernel(outputs, batch_y):
    return pl.pallas_call(
        _fc_kernel,
        grid=(_B,),
        in_specs=[
            pl.BlockSpec((1, _L, _C), lambda b: (b, 0, 0)),
            pl.BlockSpec((1, _L, _C), lambda b: (b, 0, 0)),
        ],
        out_specs=pl.BlockSpec((1, _L, _C), lambda b: (b, 0, 0)),
        out_shape=jax.ShapeDtypeStruct((_B, _L, _C), jnp.float32),
    )(outputs, batch_y)


# speedup vs baseline: 9.9216x; 1.0698x over previous
import jax
import jax.numpy as jnp
from jax.experimental import pallas as pl

_B = 32
_L = 2101
_C = 64
_S = 64
_NB = 32
_W = _NB * _S
_PAD = _L - _W


_BB = 4


def _fc_kernel(o_ref, y_ref, out_ref):
    for bi in range(_BB):
        o = o_ref[bi]
        y = y_ref[bi]
        d = o - y
        sq = d * d
        main = sq[:_W].reshape(_NB, _S, _C)
        s = jnp.sum(main, axis=1)
        tail = jnp.sum(sq[_W:], axis=0, keepdims=True)
        mp = s[:-1] + s[1:]
        nz = (mp != 0).astype(jnp.float32)
        num = jnp.concatenate([mp[:1], mp[:-1] + mp[1:], mp[-1:]], axis=0)
        cnt = jnp.concatenate([nz[:1], nz[:-1] + nz[1:], nz[-1:]], axis=0)
        v = num / cnt
        body = jnp.broadcast_to(v[:, None, :], (_NB, _S, _C)).reshape(_W, _C)
        tail_b = jnp.broadcast_to(tail, (_PAD, _C))
        out_ref[bi] = jnp.concatenate([body, tail_b], axis=0)


def kernel(outputs, batch_y):
    return pl.pallas_call(
        _fc_kernel,
        grid=(_B // _BB,),
        in_specs=[
            pl.BlockSpec((_BB, _L, _C), lambda b: (b, 0, 0)),
            pl.BlockSpec((_BB, _L, _C), lambda b: (b, 0, 0)),
        ],
        out_specs=pl.BlockSpec((_BB, _L, _C), lambda b: (b, 0, 0)),
        out_shape=jax.ShapeDtypeStruct((_B, _L, _C), jnp.float32),
    )(outputs, batch_y)
